# Initial kernel scaffold; baseline (speedup 1.0000x reference)
import jax
import jax.numpy as jnp
from jax.experimental import pallas as pl


def kernel(x, tables):
    raise NotImplementedError("write your pallas kernel here")


# speedup vs baseline: 3.7097x; 3.7097x over previous
import math

import jax
import jax.numpy as jnp
import numpy as np
from jax import lax
from jax.experimental import pallas as pl
from jax.experimental.pallas import tpu as pltpu
from jax.experimental.pallas import tpu_sc as plsc

_IN_DIM = 3
_N_LEVELS = 16
_F = 2
_LOG2_HASHMAP = 19
_BASE_RES = 16
_DESIRED_RES = 512
_N_POINTS = 524288
_PRIMES = (1, 2654435761, 805459861)

_beta = math.exp((math.log(_DESIRED_RES) - math.log(_BASE_RES)) / (_BASE_RES - 1))
_RES = [int(math.floor(_BASE_RES * _beta ** l)) for l in range(_N_LEVELS)]
_HS = [min(r ** _IN_DIM, 2 ** _LOG2_HASHMAP) for r in _RES]
_OFF = [0]
for _h in _HS:
    _OFF.append(_OFF[-1] + _h)
_TOTAL_ROWS = _OFF[-1]

_NC, _NS, _L = 2, 16, 16
_NW = _NC * _NS
_PPW = _N_POINTS // _NW
_C = 512
_NCH = _PPW // _C
_NGRP = _C // _L


def _wrap32(v):
    v &= 0xFFFFFFFF
    if v >= 1 << 31:
        v -= 1 << 32
    return jnp.int32(v)


def _tec_body(xt_hbm, tab_hbm, out_hbm, x_v, idx_v, rows_v, out_v, sem0, sem1):
    sems = (sem0, sem1)
    wid = lax.axis_index("s") * _NC + lax.axis_index("c")
    lanes = lax.iota(jnp.int32, _L)
    lanes2 = lanes * 2

    def compute_idx(l, b):
        res = float(_RES[l])
        hs = _HS[l]
        off = jnp.int32(_OFF[l])
        pow2 = hs & (hs - 1) == 0
        inv = float(np.float32(1.0) / np.float32(hs))
        hs_c = jnp.int32(hs)

        def g_body(g, carry):
            s = g * _L
            mlo, mhi = [], []
            for d in range(_IN_DIM):
                xs = x_v[d, pl.ds(s, _L)] * res
                xi = xs.astype(jnp.int32)
                if d == 0:
                    lo, hi = xi, xi + 1
                else:
                    p = _wrap32(_PRIMES[d])
                    lo = xi * p
                    hi = lo + p
                mlo.append(lo)
                mhi.append(hi)
            for j in range(8):
                h = ((mhi[0] if j & 1 else mlo[0])
                     ^ (mhi[1] if j & 2 else mlo[1])
                     ^ (mhi[2] if j & 4 else mlo[2]))
                if pow2:
                    r = h & jnp.int32(hs - 1)
                else:
                    hf = h.astype(jnp.float32)
                    hf = jnp.where(hf < 0.0, hf + 4294967296.0, hf)
                    q = (hf * inv).astype(jnp.int32)
                    r = h - q * hs_c
                    r = jnp.where(r < 0, r + hs_c, r)
                    r = jnp.where(r >= hs_c, r - hs_c, r)
                idx_v[b, pl.ds(j * _C + s, _L)] = (r + off) * 4
            return carry

        lax.fori_loop(0, _NGRP, g_body, 0)

    def combine(l, b):
        res = float(_RES[l])
        rows = rows_v.at[b]
        f0col = jnp.full((_L,), 2 * l, jnp.int32)
        f1col = f0col + 1

        def g_body(g, carry):
            s = g * _L
            xf, om = [], []
            for d in range(_IN_DIM):
                xs = x_v[d, pl.ds(s, _L)] * res
                xi = xs.astype(jnp.int32)
                f = xs - xi.astype(jnp.float32)
                xf.append(f)
                om.append(1.0 - f)
            w12 = (om[1] * om[2], xf[1] * om[2], om[1] * xf[2], xf[1] * xf[2])
            acc0 = jnp.zeros((_L,), jnp.float32)
            acc1 = jnp.zeros((_L,), jnp.float32)
            for j in range(8):
                wj = (xf[0] if j & 1 else om[0]) * w12[j >> 1]
                w0 = 2 * (j * _C + s) + lanes2
                row = w0 >> 3
                c0 = w0 & 7
                g0 = plsc.load_gather(rows, [row, c0])
                g1 = plsc.load_gather(rows, [row, c0 + 1])
                acc0 = acc0 + wj * g0
                acc1 = acc1 + wj * g1
            prow = s + lanes
            plsc.store_scatter(out_v, [prow, f0col], acc0)
            plsc.store_scatter(out_v, [prow, f1col], acc1)
            return carry

        lax.fori_loop(0, _NGRP, g_body, 0)

    def chunk_body(ch, carry):
        base = wid * _PPW + ch * _C
        pltpu.sync_copy(xt_hbm.at[:, pl.ds(base, _C)], x_v)
        copies = [None, None]
        compute_idx(0, 0)
        copies[0] = pltpu.async_copy(tab_hbm.at[idx_v.at[0]], rows_v.at[0], sems[0])
        for l in range(_N_LEVELS):
            b = l % 2
            if l + 1 < _N_LEVELS:
                b2 = (l + 1) % 2
                compute_idx(l + 1, b2)
                copies[b2] = pltpu.async_copy(
                    tab_hbm.at[idx_v.at[b2]], rows_v.at[b2], sems[b2])
            copies[b].wait()
            combine(l, b)
        pltpu.sync_copy(out_v, out_hbm.at[pl.ds(base, _C)])
        return carry

    lax.fori_loop(0, _NCH, chunk_body, 0)


def kernel(x, tables):
    xt = x.T
    mesh = plsc.VectorSubcoreMesh(
        core_axis_name="c", subcore_axis_name="s",
        num_cores=_NC, num_subcores=_NS)
    k = pl.kernel(
        _tec_body,
        out_type=jax.ShapeDtypeStruct((_N_POINTS, _N_LEVELS * _F), jnp.float32),
        mesh=mesh,
        compiler_params=pltpu.CompilerParams(
            needs_layout_passes=False, use_tc_tiling_on_sc=False),
        scratch_types=[
            pltpu.VMEM((_IN_DIM, _C), jnp.float32),
            pltpu.VMEM((2, 8 * _C), jnp.int32),
            pltpu.VMEM((2, 8 * _C, _F), jnp.float32),
            pltpu.VMEM((_C, _N_LEVELS * _F), jnp.float32),
            pltpu.SemaphoreType.DMA,
            pltpu.SemaphoreType.DMA,
        ],
    )
    return k(xt, tables)
